# baseline (speedup 1.0000x reference)
import functools

import jax
import jax.numpy as jnp
from jax import lax
from jax.experimental import pallas as pl
from jax.experimental.pallas import tpu as pltpu
from jax.experimental.pallas import tpu_sc as plsc

_NC = 2
_NS = 16
_NW = _NC * _NS
_NBUF = 2


def _build(batch, seq, hidden, chunk):
    n_per_w = batch * seq // _NW
    n_ch = n_per_w // chunk
    w_per_b = _NW // batch
    mesh = plsc.VectorSubcoreMesh(core_axis_name="c", subcore_axis_name="s")

    @functools.partial(
        pl.kernel,
        mesh=mesh,
        out_type=jax.ShapeDtypeStruct((batch, seq, hidden), jnp.float32),
        scratch_types=(
            [pltpu.VMEM((n_per_w,), jnp.int32)]
            + [pltpu.VMEM((chunk, hidden), jnp.float32) for _ in range(_NBUF)]
            + [pltpu.SemaphoreType.DMA for _ in range(2 * _NBUF)]
        ),
    )
    def emb(idx_hbm, table_hbm, out_hbm, idx_v, *rest):
        bufs = rest[:_NBUF]
        gsems = rest[_NBUF:2 * _NBUF]
        wsems = rest[2 * _NBUF:]
        wid = lax.axis_index("s") * _NC + lax.axis_index("c")
        bb = wid // w_per_b
        col = (wid % w_per_b) * n_per_w
        pltpu.sync_copy(idx_hbm.at[bb, pl.ds(col, n_per_w)], idx_v)

        def start_gather(i):
            return pltpu.async_copy(table_hbm.at[idx_v.at[pl.ds(i * chunk, chunk)]],
                                    bufs[i % _NBUF], gsems[i % _NBUF])

        lookahead = _NBUF - 1
        gathers = {j: start_gather(j) for j in range(min(lookahead, n_ch))}
        writebacks = {}
        for i in range(n_ch):
            b = i % _NBUF
            gathers.pop(i).wait()
            writebacks[i] = pltpu.async_copy(
                bufs[b], out_hbm.at[bb, pl.ds(col + i * chunk, chunk)], wsems[b])
            j = i + lookahead
            if j < n_ch:
                if i - 1 in writebacks:
                    writebacks.pop(i - 1).wait()
                gathers[j] = start_gather(j)
        for i in sorted(writebacks):
            writebacks[i].wait()

    return emb


def kernel(input, word_embeddings):
    b, s = input.shape
    v, d = word_embeddings.shape
    idx = input.astype(jnp.int32)
    return _build(b, s, d, 32)(idx, word_embeddings)


# speedup vs baseline: 1.0432x; 1.0432x over previous
import functools

import jax
import jax.numpy as jnp
from jax import lax
from jax.experimental import pallas as pl
from jax.experimental.pallas import tpu as pltpu
from jax.experimental.pallas import tpu_sc as plsc

_NC = 2
_NS = 16
_NW = _NC * _NS
_NBUF = 3


def _build(batch, seq, hidden, chunk):
    n_per_w = batch * seq // _NW
    n_ch = n_per_w // chunk
    w_per_b = _NW // batch
    mesh = plsc.VectorSubcoreMesh(core_axis_name="c", subcore_axis_name="s")

    @functools.partial(
        pl.kernel,
        mesh=mesh,
        out_type=jax.ShapeDtypeStruct((batch, seq, hidden), jnp.float32),
        scratch_types=(
            [pltpu.VMEM((n_per_w,), jnp.int32)]
            + [pltpu.VMEM((chunk, hidden), jnp.float32) for _ in range(_NBUF)]
            + [pltpu.SemaphoreType.DMA for _ in range(2 * _NBUF)]
        ),
    )
    def emb(idx_hbm, table_hbm, out_hbm, idx_v, *rest):
        bufs = rest[:_NBUF]
        gsems = rest[_NBUF:2 * _NBUF]
        wsems = rest[2 * _NBUF:]
        wid = lax.axis_index("s") * _NC + lax.axis_index("c")
        bb = wid // w_per_b
        col = (wid % w_per_b) * n_per_w
        pltpu.sync_copy(idx_hbm.at[bb, pl.ds(col, n_per_w)], idx_v)

        def start_gather(i):
            return pltpu.async_copy(table_hbm.at[idx_v.at[pl.ds(i * chunk, chunk)]],
                                    bufs[i % _NBUF], gsems[i % _NBUF])

        lookahead = _NBUF - 1
        gathers = {j: start_gather(j) for j in range(min(lookahead, n_ch))}
        writebacks = {}
        for i in range(n_ch):
            b = i % _NBUF
            gathers.pop(i).wait()
            writebacks[i] = pltpu.async_copy(
                bufs[b], out_hbm.at[bb, pl.ds(col + i * chunk, chunk)], wsems[b])
            j = i + lookahead
            if j < n_ch:
                if i - 1 in writebacks:
                    writebacks.pop(i - 1).wait()
                gathers[j] = start_gather(j)
        for i in sorted(writebacks):
            writebacks[i].wait()

    return emb


def kernel(input, word_embeddings):
    b, s = input.shape
    v, d = word_embeddings.shape
    idx = input.astype(jnp.int32)
    return _build(b, s, d, 32)(idx, word_embeddings)


# speedup vs baseline: 1.0651x; 1.0210x over previous
import functools

import jax
import jax.numpy as jnp
from jax import lax
from jax.experimental import pallas as pl
from jax.experimental.pallas import tpu as pltpu
from jax.experimental.pallas import tpu_sc as plsc

_NC = 2
_NS = 16
_NW = _NC * _NS
_NBUF = 6


def _build(batch, seq, hidden, chunk):
    n_per_w = batch * seq // _NW
    n_ch = n_per_w // chunk
    w_per_b = _NW // batch
    mesh = plsc.VectorSubcoreMesh(core_axis_name="c", subcore_axis_name="s")

    @functools.partial(
        pl.kernel,
        mesh=mesh,
        out_type=jax.ShapeDtypeStruct((batch, seq, hidden), jnp.float32),
        scratch_types=(
            [pltpu.VMEM((n_per_w,), jnp.int32)]
            + [pltpu.VMEM((chunk, hidden), jnp.float32) for _ in range(_NBUF)]
            + [pltpu.SemaphoreType.DMA for _ in range(2 * _NBUF)]
        ),
    )
    def emb(idx_hbm, table_hbm, out_hbm, idx_v, *rest):
        bufs = rest[:_NBUF]
        gsems = rest[_NBUF:2 * _NBUF]
        wsems = rest[2 * _NBUF:]
        wid = lax.axis_index("s") * _NC + lax.axis_index("c")
        bb = wid // w_per_b
        col = (wid % w_per_b) * n_per_w
        pltpu.sync_copy(idx_hbm.at[bb, pl.ds(col, n_per_w)], idx_v)

        def start_gather(i):
            return pltpu.async_copy(table_hbm.at[idx_v.at[pl.ds(i * chunk, chunk)]],
                                    bufs[i % _NBUF], gsems[i % _NBUF])

        lookahead = _NBUF - 1
        gathers = {j: start_gather(j) for j in range(min(lookahead, n_ch))}
        writebacks = {}
        for i in range(n_ch):
            b = i % _NBUF
            gathers.pop(i).wait()
            writebacks[i] = pltpu.async_copy(
                bufs[b], out_hbm.at[bb, pl.ds(col + i * chunk, chunk)], wsems[b])
            j = i + lookahead
            if j < n_ch:
                if i - 1 in writebacks:
                    writebacks.pop(i - 1).wait()
                gathers[j] = start_gather(j)
        for i in sorted(writebacks):
            writebacks[i].wait()

    return emb


def kernel(input, word_embeddings):
    b, s = input.shape
    v, d = word_embeddings.shape
    idx = input.astype(jnp.int32)
    return _build(b, s, d, 16)(idx, word_embeddings)
